# baseline (speedup 1.0000x reference)
import functools

import jax
import jax.numpy as jnp
from jax import lax
from jax.experimental import pallas as pl
from jax.experimental.pallas import tpu as pltpu
from jax.experimental.pallas import tpu_sc as plsc

_NUM_NODES = 1000000
_EMBED_DIM = 64
_BATCH = 16384

_INFO = plsc.get_sparse_core_info()
_NC = _INFO.num_cores
_NS = _INFO.num_subcores
_NW = _NC * _NS
_B_PER_W = _BATCH // _NW
_CHUNK = 128
_NCHUNK = _B_PER_W // _CHUNK
_NGROUP = _B_PER_W // 16


@functools.partial(
    pl.kernel,
    mesh=plsc.VectorSubcoreMesh(core_axis_name="c", subcore_axis_name="s"),
    out_type=jax.ShapeDtypeStruct((_BATCH // 2, 2 * _EMBED_DIM), jnp.float32),
    scratch_types=[
        pltpu.VMEM((_B_PER_W,), jnp.int32),
        pltpu.VMEM((_B_PER_W,), jnp.int32),
        pltpu.VMEM((_B_PER_W,), jnp.int32),
        pltpu.VMEM((_B_PER_W, 2 * _EMBED_DIM), jnp.float32),
        pltpu.VMEM((_B_PER_W // 2, 2 * _EMBED_DIM), jnp.float32),
        pltpu.SemaphoreType.DMA,
    ],
    compiler_params=pltpu.CompilerParams(needs_layout_passes=False),
)
def _gather_kernel(idx_hbm, w2_hbm, out_hbm, idx_v, pair_v, off_v, rows_v,
                   out_v, sem):
    wid = lax.axis_index("s") * _NC + lax.axis_index("c")
    base = wid * _B_PER_W
    pltpu.sync_copy(idx_hbm.at[pl.ds(base, _B_PER_W)], idx_v)

    def prep(g, _):
        iv = idx_v[pl.ds(g * 16, 16)]
        pair_v[pl.ds(g * 16, 16)] = lax.shift_right_logical(iv, 1)
        off_v[pl.ds(g * 16, 16)] = lax.shift_left(iv & 1, 6)
        return _

    lax.fori_loop(0, _NGROUP, prep, None)

    copies = []
    for j in range(_NCHUNK):
        copies.append(
            pltpu.make_async_copy(
                w2_hbm.at[pair_v.at[pl.ds(j * _CHUNK, _CHUNK)]],
                rows_v.at[pl.ds(j * _CHUNK, _CHUNK)],
                sem,
            )
        )
        copies[-1].start()
    for c in copies:
        c.wait()

    lanes = lax.iota(jnp.int32, 16)

    def extract(g, _):
        rowi = g * 16 + lanes
        offs = off_v[pl.ds(g * 16, 16)]
        q = lax.shift_right_logical(rowi, 1)
        segb = lax.shift_left(rowi & 1, 6)
        for c in range(_EMBED_DIM):
            v = plsc.load_gather(rows_v, [rowi, offs + c])
            plsc.store_scatter(out_v, [q, segb + c], v)
        return _

    lax.fori_loop(0, _NGROUP, extract, None)

    pltpu.sync_copy(out_v, out_hbm.at[pl.ds(wid * (_B_PER_W // 2),
                                            _B_PER_W // 2)])


def kernel(indices, weight):
    w2 = weight.reshape(_NUM_NODES // 2, 2 * _EMBED_DIM)
    out2 = _gather_kernel(indices.astype(jnp.int32), w2)
    return out2.reshape(_BATCH, _EMBED_DIM)


# speedup vs baseline: 2.6498x; 2.6498x over previous
import functools

import jax
import jax.numpy as jnp
from jax import lax
from jax.experimental import pallas as pl
from jax.experimental.pallas import tpu as pltpu
from jax.experimental.pallas import tpu_sc as plsc

_NUM_NODES = 1000000
_EMBED_DIM = 64
_BATCH = 16384

_INFO = plsc.get_sparse_core_info()
_NC = _INFO.num_cores
_NS = _INFO.num_subcores
_NW = _NC * _NS
_B_PER_W = _BATCH // _NW
_NBUF = 8


@functools.partial(
    pl.kernel,
    mesh=plsc.VectorSubcoreMesh(core_axis_name="c", subcore_axis_name="s"),
    out_type=jax.ShapeDtypeStruct((_BATCH * _EMBED_DIM,), jnp.float32),
    scratch_types=[
        pltpu.VMEM((_B_PER_W + 16,), jnp.int32),
        pltpu.VMEM((_B_PER_W * _EMBED_DIM,), jnp.float32),
    ]
    + [pltpu.VMEM((_EMBED_DIM, 128), jnp.float32) for _ in range(_NBUF)]
    + [pltpu.SemaphoreType.DMA],
    compiler_params=pltpu.CompilerParams(needs_layout_passes=False),
)
def _gather_kernel(idx_hbm, wt_hbm, out_hbm, idx_v, out_v, *blocks_and_sem):
    blocks = blocks_and_sem[:_NBUF]
    sem = blocks_and_sem[_NBUF]
    wid = lax.axis_index("s") * _NC + lax.axis_index("c")
    base = wid * _B_PER_W
    pltpu.sync_copy(idx_hbm.at[pl.ds(base, _B_PER_W)],
                    idx_v.at[pl.ds(0, _B_PER_W)])

    rows = [lax.iota(jnp.int32, 16) + 16 * k for k in range(4)]

    def group(g, _):
        jo = g * _NBUF
        iv = idx_v[pl.ds(jo, 16)]
        descs = []
        for b in range(_NBUF):
            tcol = pl.multiple_of(
                lax.shift_left(lax.shift_right_logical(iv[b], 7), 7), 128)
            d = pltpu.make_async_copy(
                wt_hbm.at[:, pl.ds(tcol, 128)], blocks[b], sem)
            d.start()
            descs.append(d)
        for b in range(_NBUF):
            descs[b].wait()
            lane = jnp.full((16,), iv[b] & 127, jnp.int32)
            j = jo + b
            for k in range(4):
                v = plsc.load_gather(blocks[b], [rows[k], lane])
                out_v[pl.ds(j * _EMBED_DIM + 16 * k, 16)] = v
        return _

    lax.fori_loop(0, _B_PER_W // _NBUF, group, None)
    pltpu.sync_copy(out_v, out_hbm.at[pl.ds(base * _EMBED_DIM,
                                            _B_PER_W * _EMBED_DIM)])


def kernel(indices, weight):
    flat = _gather_kernel(indices.astype(jnp.int32), weight.T)
    return flat.reshape(_BATCH, _EMBED_DIM)
